# baseline (speedup 1.0000x reference)
import functools

import jax
import jax.numpy as jnp
from jax import lax
from jax.experimental import pallas as pl
from jax.experimental.pallas import tpu as pltpu
from jax.experimental.pallas import tpu_sc as plsc

EMBED = 128
N_CTX = 4096 * 20
N_TGT = 4096
N_NOISE = 4096 * 20
TOTAL = N_CTX + N_TGT + N_NOISE

NC = 2
NS = 16
NW = NC * NS
B_PER_W = TOTAL // NW
CHUNK = 128
NCHUNK = B_PER_W // CHUNK

_mesh = plsc.VectorSubcoreMesh(core_axis_name="c", subcore_axis_name="s")


@functools.partial(
    pl.kernel,
    mesh=_mesh,
    out_type=jax.ShapeDtypeStruct((TOTAL, EMBED), jnp.float32),
    scratch_types=[
        pltpu.VMEM((NCHUNK, CHUNK), jnp.int32),
        pltpu.VMEM((CHUNK, EMBED), jnp.float32),
        pltpu.SemaphoreType.DMA,
    ],
)
def _gather_all(idx_hbm, table_hbm, out_hbm, idx_v, rows_v, sem):
    wid = lax.axis_index("s") * NC + lax.axis_index("c")
    pltpu.sync_copy(idx_hbm.at[wid], idx_v)

    def body(j, carry):
        pltpu.async_copy(table_hbm.at[idx_v.at[j]], rows_v, sem).wait()
        pltpu.sync_copy(
            rows_v, out_hbm.at[pl.ds(wid * B_PER_W + j * CHUNK, CHUNK)]
        )
        return carry

    lax.fori_loop(0, NCHUNK, body, 0)


def kernel(context_ids, target_ids, noise_ids, embeddings):
    idx = jnp.concatenate(
        [
            context_ids.reshape(-1).astype(jnp.int32),
            target_ids.reshape(-1).astype(jnp.int32),
            noise_ids.reshape(-1).astype(jnp.int32),
        ]
    ).reshape(NW, NCHUNK, CHUNK)
    flat = _gather_all(idx, embeddings)
    ctx = flat[:N_CTX].reshape(4096, 20, EMBED)
    tgt = flat[N_CTX : N_CTX + N_TGT]
    noise = flat[N_CTX + N_TGT :].reshape(4096, 20, EMBED)
    return (ctx, tgt, noise)


# speedup vs baseline: 1.1136x; 1.1136x over previous
import functools

import jax
import jax.numpy as jnp
from jax import lax
from jax.experimental import pallas as pl
from jax.experimental.pallas import tpu as pltpu
from jax.experimental.pallas import tpu_sc as plsc

EMBED = 128
N_CTX = 4096 * 20
N_TGT = 4096
N_NOISE = 4096 * 20
TOTAL = N_CTX + N_TGT + N_NOISE

NC = 2
NS = 16
NW = NC * NS
B_PER_W = TOTAL // NW
CHUNK = 128
NCHUNK = B_PER_W // CHUNK

_mesh = plsc.VectorSubcoreMesh(core_axis_name="c", subcore_axis_name="s")


NBUF = 4


@functools.partial(
    pl.kernel,
    mesh=_mesh,
    out_type=jax.ShapeDtypeStruct((TOTAL, EMBED), jnp.float32),
    scratch_types=[
        pltpu.VMEM((NCHUNK, CHUNK), jnp.int32),
        pltpu.VMEM((NBUF, CHUNK, EMBED), jnp.float32),
        pltpu.SemaphoreType.DMA,
        pltpu.SemaphoreType.DMA,
    ],
)
def _gather_all(idx_hbm, table_hbm, out_hbm, idx_v, rows_v, sem_g, sem_s):
    wid = lax.axis_index("s") * NC + lax.axis_index("c")
    base = wid * B_PER_W
    pltpu.sync_copy(idx_hbm.at[wid], idx_v)

    def start_gather(j):
        pltpu.async_copy(
            table_hbm.at[idx_v.at[j]], rows_v.at[lax.rem(j, NBUF)], sem_g
        )

    for j in range(NBUF - 1):
        start_gather(jnp.int32(j))

    def body(j, carry):
        buf = lax.rem(j, NBUF)
        pltpu.make_async_copy(
            table_hbm.at[idx_v.at[j]], rows_v.at[buf], sem_g
        ).wait()
        pltpu.async_copy(
            rows_v.at[buf], out_hbm.at[pl.ds(base + j * CHUNK, CHUNK)], sem_s
        )
        @pl.when(j > 0)
        def _():
            pltpu.make_async_copy(
                rows_v.at[buf], out_hbm.at[pl.ds(base, CHUNK)], sem_s
            ).wait()

        @pl.when(j + NBUF - 1 < NCHUNK)
        def _():
            start_gather(j + NBUF - 1)

        return carry

    lax.fori_loop(0, NCHUNK, body, 0)
    pltpu.make_async_copy(
        rows_v.at[0], out_hbm.at[pl.ds(base, CHUNK)], sem_s
    ).wait()


def kernel(context_ids, target_ids, noise_ids, embeddings):
    idx = jnp.concatenate(
        [
            context_ids.reshape(-1).astype(jnp.int32),
            target_ids.reshape(-1).astype(jnp.int32),
            noise_ids.reshape(-1).astype(jnp.int32),
        ]
    ).reshape(NW, NCHUNK, CHUNK)
    flat = _gather_all(idx, embeddings)
    ctx = flat[:N_CTX].reshape(4096, 20, EMBED)
    tgt = flat[N_CTX : N_CTX + N_TGT]
    noise = flat[N_CTX + N_TGT :].reshape(4096, 20, EMBED)
    return (ctx, tgt, noise)


# speedup vs baseline: 1.3688x; 1.2292x over previous
import functools

import jax
import jax.numpy as jnp
from jax import lax
from jax.experimental import pallas as pl
from jax.experimental.pallas import tpu as pltpu
from jax.experimental.pallas import tpu_sc as plsc

EMBED = 128
N_CTX = 4096 * 20
N_TGT = 4096
N_NOISE = 4096 * 20
TOTAL = N_CTX + N_TGT + N_NOISE

NC = 2
NS = 16
NW = NC * NS
CHUNK = 128
NCHUNK_TOTAL = TOTAL // CHUNK
NCHUNK_W = NCHUNK_TOTAL // NW
CTX_CHUNKS = N_CTX // CHUNK
TGT_CHUNKS = N_TGT // CHUNK
NBUF = 4

_mesh = plsc.VectorSubcoreMesh(core_axis_name="c", subcore_axis_name="s")


@functools.partial(
    pl.kernel,
    mesh=_mesh,
    out_type=(
        jax.ShapeDtypeStruct((N_CTX, EMBED), jnp.float32),
        jax.ShapeDtypeStruct((N_TGT, EMBED), jnp.float32),
        jax.ShapeDtypeStruct((N_NOISE, EMBED), jnp.float32),
    ),
    scratch_types=[
        pltpu.VMEM((NCHUNK_W, CHUNK), jnp.int32),
        pltpu.VMEM((NBUF, CHUNK, EMBED), jnp.float32),
        pltpu.SemaphoreType.DMA,
        pltpu.SemaphoreType.DMA,
    ],
)
def _gather_all(idx_hbm, table_hbm, ctx_hbm, tgt_hbm, noise_hbm,
                idx_v, rows_v, sem_g, sem_s):
    wid = lax.axis_index("s") * NC + lax.axis_index("c")
    base_chunk = wid * NCHUNK_W
    pltpu.sync_copy(idx_hbm.at[wid], idx_v)

    def start_gather(j):
        pltpu.async_copy(
            table_hbm.at[idx_v.at[j]], rows_v.at[lax.rem(j, NBUF)], sem_g
        )

    for j in range(NBUF - 1):
        start_gather(jnp.int32(j))

    def body(j, carry):
        buf = lax.rem(j, NBUF)
        g = base_chunk + j
        pltpu.make_async_copy(
            table_hbm.at[idx_v.at[j]], rows_v.at[buf], sem_g
        ).wait()

        @pl.when(g < CTX_CHUNKS)
        def _():
            pltpu.async_copy(
                rows_v.at[buf], ctx_hbm.at[pl.ds(g * CHUNK, CHUNK)], sem_s
            )

        @pl.when(jnp.logical_and(g >= CTX_CHUNKS, g < CTX_CHUNKS + TGT_CHUNKS))
        def _():
            pltpu.async_copy(
                rows_v.at[buf],
                tgt_hbm.at[pl.ds((g - CTX_CHUNKS) * CHUNK, CHUNK)],
                sem_s,
            )

        @pl.when(g >= CTX_CHUNKS + TGT_CHUNKS)
        def _():
            pltpu.async_copy(
                rows_v.at[buf],
                noise_hbm.at[pl.ds((g - CTX_CHUNKS - TGT_CHUNKS) * CHUNK, CHUNK)],
                sem_s,
            )

        @pl.when(j > 0)
        def _():
            pltpu.make_async_copy(
                rows_v.at[buf], ctx_hbm.at[pl.ds(0, CHUNK)], sem_s
            ).wait()

        @pl.when(j + NBUF - 1 < NCHUNK_W)
        def _():
            start_gather(j + NBUF - 1)

        return carry

    lax.fori_loop(0, NCHUNK_W, body, 0)
    pltpu.make_async_copy(
        rows_v.at[0], ctx_hbm.at[pl.ds(0, CHUNK)], sem_s
    ).wait()


def kernel(context_ids, target_ids, noise_ids, embeddings):
    idx = jnp.concatenate(
        [
            context_ids.reshape(-1).astype(jnp.int32),
            target_ids.reshape(-1).astype(jnp.int32),
            noise_ids.reshape(-1).astype(jnp.int32),
        ]
    ).reshape(NW, NCHUNK_W, CHUNK)
    ctx, tgt, noise = _gather_all(idx, embeddings)
    return (
        ctx.reshape(4096, 20, EMBED),
        tgt,
        noise.reshape(4096, 20, EMBED),
    )


# speedup vs baseline: 3.9045x; 2.8525x over previous
import functools

import jax
import jax.numpy as jnp
from jax import lax
from jax.experimental import pallas as pl
from jax.experimental.pallas import tpu as pltpu
from jax.experimental.pallas import tpu_sc as plsc

EMBED = 128
N_CTX = 4096 * 20
N_TGT = 4096
N_NOISE = 4096 * 20
TOTAL = N_CTX + N_TGT + N_NOISE

NC = 2
NS = 16
NW = NC * NS
CHUNK = 128
NCHUNK_TOTAL = TOTAL // CHUNK
NCHUNK_W = NCHUNK_TOTAL // NW
CTX_CHUNKS = N_CTX // CHUNK
TGT_CHUNKS = N_TGT // CHUNK
NBUF = 4

_mesh = plsc.VectorSubcoreMesh(core_axis_name="c", subcore_axis_name="s")


@functools.partial(
    pl.kernel,
    mesh=_mesh,
    out_type=(
        jax.ShapeDtypeStruct((N_CTX, EMBED), jnp.float32),
        jax.ShapeDtypeStruct((N_TGT, EMBED), jnp.float32),
        jax.ShapeDtypeStruct((N_NOISE, EMBED), jnp.float32),
    ),
    scratch_types=[
        pltpu.VMEM((NCHUNK_W, CHUNK), jnp.int32),
        pltpu.VMEM((NBUF, CHUNK, EMBED), jnp.float32),
        pltpu.SemaphoreType.DMA,
        pltpu.SemaphoreType.DMA,
    ],
)
def _gather_all(idx_hbm, table_hbm, ctx_hbm, tgt_hbm, noise_hbm,
                idx_v, rows_v, sem_g, sem_s):
    wid = lax.axis_index("s") * NC + lax.axis_index("c")
    base_chunk = wid * NCHUNK_W
    pltpu.sync_copy(idx_hbm.at[wid], idx_v)

    def start_gather(j):
        pltpu.async_copy(
            table_hbm.at[idx_v.at[j]], rows_v.at[lax.rem(j, NBUF)], sem_g
        )

    for j in range(NBUF - 1):
        start_gather(jnp.int32(j))

    def body(j, carry):
        buf = lax.rem(j, NBUF)
        g = base_chunk + j
        pltpu.make_async_copy(
            table_hbm.at[idx_v.at[j]], rows_v.at[buf], sem_g
        ).wait()

        @pl.when(g < CTX_CHUNKS)
        def _():
            pltpu.async_copy(
                rows_v.at[buf], ctx_hbm.at[pl.ds(g * CHUNK, CHUNK)], sem_s
            )

        @pl.when(jnp.logical_and(g >= CTX_CHUNKS, g < CTX_CHUNKS + TGT_CHUNKS))
        def _():
            pltpu.async_copy(
                rows_v.at[buf],
                tgt_hbm.at[pl.ds((g - CTX_CHUNKS) * CHUNK, CHUNK)],
                sem_s,
            )

        @pl.when(g >= CTX_CHUNKS + TGT_CHUNKS)
        def _():
            pltpu.async_copy(
                rows_v.at[buf],
                noise_hbm.at[pl.ds((g - CTX_CHUNKS - TGT_CHUNKS) * CHUNK, CHUNK)],
                sem_s,
            )

        @pl.when(j > 0)
        def _():
            pltpu.make_async_copy(
                rows_v.at[buf], ctx_hbm.at[pl.ds(0, CHUNK)], sem_s
            ).wait()

        @pl.when(j + NBUF - 1 < NCHUNK_W)
        def _():
            start_gather(j + NBUF - 1)

        return carry

    lax.fori_loop(0, NCHUNK_W, body, 0)
    pltpu.make_async_copy(
        rows_v.at[0], ctx_hbm.at[pl.ds(0, CHUNK)], sem_s
    ).wait()


def kernel(context_ids, target_ids, noise_ids, embeddings):
    idx = jnp.concatenate(
        [
            context_ids.T.reshape(-1).astype(jnp.int32),
            target_ids.reshape(-1).astype(jnp.int32),
            noise_ids.T.reshape(-1).astype(jnp.int32),
        ]
    ).reshape(NW, NCHUNK_W, CHUNK)
    ctx, tgt, noise = _gather_all(idx, embeddings)
    return (
        ctx.reshape(20, 4096, EMBED).transpose(1, 0, 2),
        tgt,
        noise.reshape(20, 4096, EMBED).transpose(1, 0, 2),
    )
